# baseline (speedup 1.0000x reference)
import functools

import jax
import jax.numpy as jnp
from jax import lax
from jax.experimental import pallas as pl
from jax.experimental.pallas import tpu as pltpu
from jax.experimental.pallas import tpu_sc as plsc

T, N, D, H, E, G, C = 4, 10000, 128, 128, 320000, 16, 16
NC, NS = 2, 16
NW = NC * NS
N_PAD = 10240
RPT = N_PAD // NS
CH = 128
NCH = 80
EPT = CH * NCH
E_PAD = NW * EPT
BN = 2048
NB = N_PAD // BN

def _mesh():
    return plsc.VectorSubcoreMesh(core_axis_name="c", subcore_axis_name="s",
                                  num_cores=NC, num_subcores=NS)


def _sc_deg_body(dst_hbm, out_hbm, idx_v, zbuf, ones_v, acc):
    cid = lax.axis_index("c")
    sid = lax.axis_index("s")
    w = cid * NS + sid

    def _fill(val, ref, n):
        def body(i, _):
            ref[pl.ds(i * 16, 16)] = jnp.full((16,), val, jnp.float32)
            return 0
        lax.fori_loop(0, n // 16, body, 0)

    _fill(0.0, zbuf, RPT)
    _fill(1.0, ones_v, CH)

    for t in range(T):
        pltpu.sync_copy(zbuf, acc.at[pl.ds(sid * RPT, RPT)])
        pltpu.sync_copy(dst_hbm.at[pl.ds((t * NW + w) * NCH, NCH)], idx_v)
        plsc.subcore_barrier()

        def body(c, _):
            pltpu.sync_copy(ones_v, acc.at[idx_v.at[c]], add=True)
            return 0
        lax.fori_loop(0, NCH, body, 0)
        plsc.subcore_barrier()
        pltpu.sync_copy(
            acc.at[pl.ds(sid * RPT, RPT)],
            out_hbm.at[pl.ds((t * 2 + cid) * N_PAD + sid * RPT, RPT)],
        )
        plsc.subcore_barrier()


@functools.cache
def _sc_deg():
    return pl.kernel(
        _sc_deg_body,
        out_type=jax.ShapeDtypeStruct((T * 2 * N_PAD,), jnp.float32),
        mesh=_mesh(),
        scratch_types=[
            pltpu.VMEM((NCH, CH), jnp.int32),
            pltpu.VMEM((RPT,), jnp.float32),
            pltpu.VMEM((CH,), jnp.float32),
            pltpu.VMEM_SHARED((N_PAD,), jnp.float32),
        ],
    )


CR = 1
NCH2 = NCH // 2
NCK = NCH2 // CR


def _sc_scatter_body(y_hbm, src_hbm, dst_hbm, out_hbm, sidx, didx, bufa, bufb,
                     acc, sga, sgb):
    cid = lax.axis_index("c")
    sid = lax.axis_index("s")
    w = cid * NS + sid
    def _idx(ref, c):
        return ref.at[c] if CR == 1 else ref.at[pl.ds(c * CR, CR)]

    def gat(c, buf, sem):
        pltpu.async_copy(y_hbm.at[_idx(sidx, c)], buf, sem)

    def scat(c, buf):
        pltpu.sync_copy(buf, acc.at[_idx(didx, c)], add=True)

    def wait(buf, sem):
        pltpu.make_async_copy(y_hbm.at[pl.ds(0, CR * CH)], buf, sem).wait()

    def pipeline():
        gat(0, bufa, sga)

        def body(i, _):
            ca = 2 * i
            gat(ca + 1, bufb, sgb)
            wait(bufa, sga)
            scat(ca, bufa)

            @pl.when(ca + 2 < NCK)
            def _():
                gat(ca + 2, bufa, sga)
            wait(bufb, sgb)
            scat(ca + 1, bufb)
            return 0
        lax.fori_loop(0, NCK // 2, body, 0)

    for t in range(T):
        pltpu.sync_copy(
            y_hbm.at[pl.ds(t * N_PAD + sid * RPT, RPT)],
            acc.at[pl.ds(sid * RPT, RPT)],
        )
        pltpu.sync_copy(src_hbm.at[pl.ds((t * NW + w) * NCH, NCH2)], sidx)
        pltpu.sync_copy(dst_hbm.at[pl.ds((t * NW + w) * NCH, NCH2)], didx)
        plsc.subcore_barrier()
        pipeline()
        pltpu.sync_copy(src_hbm.at[pl.ds((t * NW + w) * NCH + NCH2, NCH2)],
                        sidx)
        pltpu.sync_copy(dst_hbm.at[pl.ds((t * NW + w) * NCH + NCH2, NCH2)],
                        didx)
        pipeline()
        plsc.subcore_barrier()
        pltpu.sync_copy(
            acc.at[pl.ds(sid * RPT, RPT)],
            out_hbm.at[pl.ds((t * 2 + cid) * N_PAD + sid * RPT, RPT)],
        )
        plsc.subcore_barrier()


@functools.cache
def _sc_scatter():
    return pl.kernel(
        _sc_scatter_body,
        out_type=jax.ShapeDtypeStruct((T * 2 * N_PAD, H), jnp.float32),
        mesh=_mesh(),
        scratch_types=[
            pltpu.VMEM((NCH2, CH), jnp.int32),
            pltpu.VMEM((NCH2, CH), jnp.int32),
            pltpu.VMEM((CR * CH, H), jnp.float32),
            pltpu.VMEM((CR * CH, H), jnp.float32),
            pltpu.VMEM_SHARED((N_PAD, H), jnp.float32),
            pltpu.SemaphoreType.DMA,
            pltpu.SemaphoreType.DMA,
        ],
    )


def _k1a_body(x_ref, w1_ref, xw_ref):
    xw_ref[...] = jnp.dot(x_ref[...], w1_ref[...],
                          preferred_element_type=jnp.float32)


def _tc_k1a(x_flat, W1):
    return pl.pallas_call(
        _k1a_body,
        grid=(T, NB),
        in_specs=[
            pl.BlockSpec((BN, D), lambda t, nb: (t * NB + nb, 0)),
            pl.BlockSpec((D, H), lambda t, nb: (0, 0)),
        ],
        out_specs=pl.BlockSpec((BN, H), lambda t, nb: (t * NB + nb, 0)),
        out_shape=jax.ShapeDtypeStruct((T * N_PAD, H), jnp.float32),
    )(x_flat, W1)


def _k1b_body(xw_ref, d0_ref, d1_ref, y_ref, dis_ref):
    deg = d0_ref[...] + d1_ref[...] + 1.0
    dis = lax.rsqrt(jnp.maximum(deg, 1.0))
    y_ref[...] = xw_ref[...] * dis
    dis_ref[...] = dis


def _tc_k1b(xw, deg_flat):
    return pl.pallas_call(
        _k1b_body,
        grid=(T, NB),
        in_specs=[
            pl.BlockSpec((BN, H), lambda t, nb: (t * NB + nb, 0)),
            pl.BlockSpec((BN, 1), lambda t, nb: ((t * 2 + 0) * NB + nb, 0)),
            pl.BlockSpec((BN, 1), lambda t, nb: ((t * 2 + 1) * NB + nb, 0)),
        ],
        out_specs=[
            pl.BlockSpec((BN, H), lambda t, nb: (t * NB + nb, 0)),
            pl.BlockSpec((BN, 1), lambda t, nb: (t * NB + nb, 0)),
        ],
        out_shape=[
            jax.ShapeDtypeStruct((T * N_PAD, H), jnp.float32),
            jax.ShapeDtypeStruct((T * N_PAD, 1), jnp.float32),
        ],
    )(xw, deg_flat, deg_flat)


def _k2_body(y_ref, p0_ref, p1_ref, dis_ref, b1_ref, w2_ref, y2_ref):
    dis = dis_ref[...]
    h = (p0_ref[...] + p1_ref[...] - y_ref[...]) * dis + b1_ref[...]
    h = jnp.maximum(h, 0.0)
    y2_ref[...] = jnp.dot(h, w2_ref[...],
                          preferred_element_type=jnp.float32) * dis


def _tc_k2(y1, p_flat, dis, b1, W2):
    return pl.pallas_call(
        _k2_body,
        grid=(T, NB),
        in_specs=[
            pl.BlockSpec((BN, H), lambda t, nb: (t * NB + nb, 0)),
            pl.BlockSpec((BN, H), lambda t, nb: ((t * 2 + 0) * NB + nb, 0)),
            pl.BlockSpec((BN, H), lambda t, nb: ((t * 2 + 1) * NB + nb, 0)),
            pl.BlockSpec((BN, 1), lambda t, nb: (t * NB + nb, 0)),
            pl.BlockSpec((1, H), lambda t, nb: (0, 0)),
            pl.BlockSpec((H, H), lambda t, nb: (0, 0)),
        ],
        out_specs=pl.BlockSpec((BN, H), lambda t, nb: (t * NB + nb, 0)),
        out_shape=jax.ShapeDtypeStruct((T * N_PAD, H), jnp.float32),
    )(y1, p_flat, p_flat, dis, b1, W2)


def _gru_cell(xt, h, Wih, Whh, bih, bhh):
    gi = lax.dot_general(xt, Wih, (((1,), (1,)), ((), ())),
                         preferred_element_type=jnp.float32) + bih
    gh = lax.dot_general(h, Whh, (((1,), (1,)), ((), ())),
                         preferred_element_type=jnp.float32) + bhh
    r = jax.nn.sigmoid(gi[:, :H] + gh[:, :H])
    z = jax.nn.sigmoid(gi[:, H:2 * H] + gh[:, H:2 * H])
    n = jnp.tanh(gi[:, 2 * H:] + r * gh[:, 2 * H:])
    return (1.0 - z) * n + z * h


def _k3_body(y_ref, q0_ref, q1_ref, dis_ref, b2_ref, batch_ref,
             wih0_ref, whh0_ref, bih0_ref, bhh0_ref,
             wih1_ref, whh1_ref, bih1_ref, bhh1_ref,
             wc_ref, bc_ref, out_ref, acc_ref, cnt_ref):
    t = pl.program_id(0)
    nb = pl.program_id(1)

    @pl.when(jnp.logical_and(t == 0, nb == 0))
    def _init():
        acc_ref[...] = jnp.zeros_like(acc_ref)
        cnt_ref[...] = jnp.zeros_like(cnt_ref)

    dis = dis_ref[...]
    h2 = (q0_ref[...] + q1_ref[...] - y_ref[...]) * dis + b2_ref[...]
    h2 = jnp.maximum(h2, 0.0)
    bvals = batch_ref[0, 0, :]
    onehot = (bvals[:, None] == lax.broadcasted_iota(jnp.int32, (1, G), 1)
              ).astype(jnp.float32)
    pool = lax.dot_general(onehot, h2, (((0,), (0,)), ((), ())),
                           preferred_element_type=jnp.float32)
    acc_ref[pl.ds(t, 1)] = acc_ref[pl.ds(t, 1)] + pool[None]

    @pl.when(t == 0)
    def _cnt():
        ones = jnp.ones((BN, 1), jnp.float32)
        cnt_ref[...] += lax.dot_general(
            onehot, ones, (((0,), (0,)), ((), ())),
            preferred_element_type=jnp.float32)

    @pl.when(jnp.logical_and(t == T - 1, nb == NB - 1))
    def _final():
        cnt = jnp.maximum(cnt_ref[...], 1.0)
        seq = acc_ref[...] / cnt[None]
        h = jnp.zeros((G, H), jnp.float32)
        seq1 = []
        for tt in range(T):
            h = _gru_cell(seq[tt], h, wih0_ref[...], whh0_ref[...],
                          bih0_ref[...], bhh0_ref[...])
            seq1.append(h)
        h = jnp.zeros((G, H), jnp.float32)
        for tt in range(T):
            h = _gru_cell(seq1[tt], h, wih1_ref[...], whh1_ref[...],
                          bih1_ref[...], bhh1_ref[...])
        out_ref[...] = jnp.dot(h, wc_ref[...],
                               preferred_element_type=jnp.float32) + bc_ref[...]


def _tc_k3(y2, q_flat, dis, b2, batch_r, Wih0, Whh0, bih0, bhh0,
           Wih1, Whh1, bih1, bhh1, Wc, bc):
    full = lambda t, nb: (0, 0)
    return pl.pallas_call(
        _k3_body,
        grid=(T, NB),
        in_specs=[
            pl.BlockSpec((BN, H), lambda t, nb: (t * NB + nb, 0)),
            pl.BlockSpec((BN, H), lambda t, nb: ((t * 2 + 0) * NB + nb, 0)),
            pl.BlockSpec((BN, H), lambda t, nb: ((t * 2 + 1) * NB + nb, 0)),
            pl.BlockSpec((BN, 1), lambda t, nb: (t * NB + nb, 0)),
            pl.BlockSpec((1, H), full),
            pl.BlockSpec((1, 1, BN), lambda t, nb: (nb, 0, 0)),
            pl.BlockSpec((3 * H, H), full),
            pl.BlockSpec((3 * H, H), full),
            pl.BlockSpec((1, 3 * H), full),
            pl.BlockSpec((1, 3 * H), full),
            pl.BlockSpec((3 * H, H), full),
            pl.BlockSpec((3 * H, H), full),
            pl.BlockSpec((1, 3 * H), full),
            pl.BlockSpec((1, 3 * H), full),
            pl.BlockSpec((H, C), full),
            pl.BlockSpec((1, C), full),
        ],
        out_specs=pl.BlockSpec((G, C), full),
        out_shape=jax.ShapeDtypeStruct((G, C), jnp.float32),
        scratch_shapes=[
            pltpu.VMEM((T, G, H), jnp.float32),
            pltpu.VMEM((G, 1), jnp.float32),
        ],
    )(y2, q_flat, q_flat, dis, b2, batch_r, Wih0, Whh0, bih0, bhh0,
      Wih1, Whh1, bih1, bhh1, Wc, bc)


def kernel(x, edge_index, batch, W1, b1, W2, b2, Wih0, Whh0, bih0, bhh0,
           Wih1, Whh1, bih1, bhh1, Wc, bc):
    i32 = jnp.int32
    x_flat = jnp.pad(x, ((0, 0), (0, N_PAD - N), (0, 0))).reshape(T * N_PAD, D)

    src = edge_index[:, 0, :]
    dst = edge_index[:, 1, :]
    fill = (N + jnp.arange(E_PAD - E, dtype=i32) % (N_PAD - N))[None, :]
    fill = jnp.broadcast_to(fill, (T, E_PAD - E))
    srcp = jnp.concatenate([src, fill], axis=1)
    dstp = jnp.concatenate([dst, fill], axis=1)
    toff = (jnp.arange(T, dtype=i32) * N_PAD)[:, None]
    srcf = (srcp + toff).reshape(T * NW * NCH, CH)
    dstf = dstp.reshape(T * NW * NCH, CH)
    batch_r = jnp.pad(batch, (0, N_PAD - N),
                      constant_values=G).reshape(NB, 1, BN)

    xw = _tc_k1a(x_flat, W1)
    deg_flat = _sc_deg()(dstf).reshape(T * 2 * N_PAD, 1)
    y1, dis = _tc_k1b(xw, deg_flat)
    p_flat = _sc_scatter()(y1, srcf, dstf)
    y2 = _tc_k2(y1, p_flat, dis, b1.reshape(1, H), W2)
    q_flat = _sc_scatter()(y2, srcf, dstf)
    return _tc_k3(y2, q_flat, dis, b2.reshape(1, H), batch_r,
                  Wih0, Whh0, bih0.reshape(1, -1), bhh0.reshape(1, -1),
                  Wih1, Whh1, bih1.reshape(1, -1), bhh1.reshape(1, -1),
                  Wc, bc.reshape(1, -1))


# speedup vs baseline: 1.0042x; 1.0042x over previous
import functools

import jax
import jax.numpy as jnp
from jax import lax
from jax.experimental import pallas as pl
from jax.experimental.pallas import tpu as pltpu
from jax.experimental.pallas import tpu_sc as plsc

T, N, D, H, E, G, C = 4, 10000, 128, 128, 320000, 16, 16
NC, NS = 2, 16
NW = NC * NS
N_PAD = 10240
RPT = N_PAD // NS
CH = 128
NCH = 80
EPT = CH * NCH
E_PAD = NW * EPT
BN = 2048
NB = N_PAD // BN

def _mesh():
    return plsc.VectorSubcoreMesh(core_axis_name="c", subcore_axis_name="s",
                                  num_cores=NC, num_subcores=NS)


def _sc_deg_body(dst_hbm, out_hbm, idx_v, zbuf, ones_v, acc):
    cid = lax.axis_index("c")
    sid = lax.axis_index("s")
    w = cid * NS + sid

    def _fill(val, ref, n):
        def body(i, _):
            ref[pl.ds(i * 16, 16)] = jnp.full((16,), val, jnp.float32)
            return 0
        lax.fori_loop(0, n // 16, body, 0)

    _fill(0.0, zbuf, RPT)
    _fill(1.0, ones_v, CH)

    for t in range(T):
        pltpu.sync_copy(zbuf, acc.at[pl.ds(sid * RPT, RPT)])
        pltpu.sync_copy(dst_hbm.at[pl.ds((t * NW + w) * NCH, NCH)], idx_v)
        plsc.subcore_barrier()

        def body(c, _):
            pltpu.sync_copy(ones_v, acc.at[idx_v.at[c]], add=True)
            return 0
        lax.fori_loop(0, NCH, body, 0)
        plsc.subcore_barrier()
        pltpu.sync_copy(
            acc.at[pl.ds(sid * RPT, RPT)],
            out_hbm.at[pl.ds((t * 2 + cid) * N_PAD + sid * RPT, RPT)],
        )
        plsc.subcore_barrier()


@functools.cache
def _sc_deg():
    return pl.kernel(
        _sc_deg_body,
        out_type=jax.ShapeDtypeStruct((T * 2 * N_PAD,), jnp.float32),
        mesh=_mesh(),
        scratch_types=[
            pltpu.VMEM((NCH, CH), jnp.int32),
            pltpu.VMEM((RPT,), jnp.float32),
            pltpu.VMEM((CH,), jnp.float32),
            pltpu.VMEM_SHARED((N_PAD,), jnp.float32),
        ],
    )


CR = 1
NCH2 = NCH // 2
NCK = NCH2 // CR


def _sc_scatter_body(y_hbm, src_hbm, dst_hbm, out_hbm, sidx, didx, bufa, bufb,
                     acc, sga, sgb):
    cid = lax.axis_index("c")
    sid = lax.axis_index("s")
    w = cid * NS + sid
    def _idx(ref, c):
        return ref.at[c] if CR == 1 else ref.at[pl.ds(c * CR, CR)]

    def gat(c, buf, sem):
        pltpu.async_copy(y_hbm.at[_idx(sidx, c)], buf, sem)

    def scat(c, buf):
        pltpu.sync_copy(buf, acc.at[_idx(didx, c)], add=True)

    def wait(buf, sem):
        pltpu.make_async_copy(y_hbm.at[pl.ds(0, CR * CH)], buf, sem).wait()

    def pipeline():
        gat(0, bufa, sga)

        def body(i, _):
            ca = 2 * i
            gat(ca + 1, bufb, sgb)
            wait(bufa, sga)
            scat(ca, bufa)

            @pl.when(ca + 2 < NCK)
            def _():
                gat(ca + 2, bufa, sga)
            wait(bufb, sgb)
            scat(ca + 1, bufb)
            return 0
        lax.fori_loop(0, NCK // 2, body, 0)

    for t in range(T):
        pltpu.sync_copy(
            y_hbm.at[pl.ds(t * N_PAD + sid * RPT, RPT)],
            acc.at[pl.ds(sid * RPT, RPT)],
        )
        pltpu.sync_copy(src_hbm.at[pl.ds((t * NW + w) * NCH, NCH2)], sidx)
        pltpu.sync_copy(dst_hbm.at[pl.ds((t * NW + w) * NCH, NCH2)], didx)
        plsc.subcore_barrier()
        pipeline()
        pltpu.sync_copy(src_hbm.at[pl.ds((t * NW + w) * NCH + NCH2, NCH2)],
                        sidx)
        pltpu.sync_copy(dst_hbm.at[pl.ds((t * NW + w) * NCH + NCH2, NCH2)],
                        didx)
        pipeline()
        plsc.subcore_barrier()
        pltpu.sync_copy(
            acc.at[pl.ds(sid * RPT, RPT)],
            out_hbm.at[pl.ds((t * 2 + cid) * N_PAD + sid * RPT, RPT)],
        )
        plsc.subcore_barrier()


@functools.cache
def _sc_scatter():
    return pl.kernel(
        _sc_scatter_body,
        out_type=jax.ShapeDtypeStruct((T * 2 * N_PAD, H), jnp.float32),
        mesh=_mesh(),
        scratch_types=[
            pltpu.VMEM((NCH2, CH), jnp.int32),
            pltpu.VMEM((NCH2, CH), jnp.int32),
            pltpu.VMEM((CR * CH, H), jnp.float32),
            pltpu.VMEM((CR * CH, H), jnp.float32),
            pltpu.VMEM_SHARED((N_PAD, H), jnp.float32),
            pltpu.SemaphoreType.DMA,
            pltpu.SemaphoreType.DMA,
        ],
    )


def _k1_body(x_ref, d0_ref, d1_ref, w1_ref, y_ref, dis_ref):
    deg = d0_ref[...] + d1_ref[...] + 1.0
    dis = lax.rsqrt(jnp.maximum(deg, 1.0))
    xw = jnp.dot(x_ref[...], w1_ref[...], preferred_element_type=jnp.float32)
    y_ref[...] = xw * dis
    dis_ref[...] = dis


def _tc_k1(x_flat, deg_flat, W1):
    return pl.pallas_call(
        _k1_body,
        grid=(T, NB),
        in_specs=[
            pl.BlockSpec((BN, D), lambda t, nb: (t * NB + nb, 0)),
            pl.BlockSpec((BN, 1), lambda t, nb: ((t * 2 + 0) * NB + nb, 0)),
            pl.BlockSpec((BN, 1), lambda t, nb: ((t * 2 + 1) * NB + nb, 0)),
            pl.BlockSpec((D, H), lambda t, nb: (0, 0)),
        ],
        out_specs=[
            pl.BlockSpec((BN, H), lambda t, nb: (t * NB + nb, 0)),
            pl.BlockSpec((BN, 1), lambda t, nb: (t * NB + nb, 0)),
        ],
        out_shape=[
            jax.ShapeDtypeStruct((T * N_PAD, H), jnp.float32),
            jax.ShapeDtypeStruct((T * N_PAD, 1), jnp.float32),
        ],
    )(x_flat, deg_flat, deg_flat, W1)


def _k2_body(y_ref, p0_ref, p1_ref, dis_ref, b1_ref, w2_ref, y2_ref):
    dis = dis_ref[...]
    h = (p0_ref[...] + p1_ref[...] - y_ref[...]) * dis + b1_ref[...]
    h = jnp.maximum(h, 0.0)
    y2_ref[...] = jnp.dot(h, w2_ref[...],
                          preferred_element_type=jnp.float32) * dis


def _tc_k2(y1, p_flat, dis, b1, W2):
    return pl.pallas_call(
        _k2_body,
        grid=(T, NB),
        in_specs=[
            pl.BlockSpec((BN, H), lambda t, nb: (t * NB + nb, 0)),
            pl.BlockSpec((BN, H), lambda t, nb: ((t * 2 + 0) * NB + nb, 0)),
            pl.BlockSpec((BN, H), lambda t, nb: ((t * 2 + 1) * NB + nb, 0)),
            pl.BlockSpec((BN, 1), lambda t, nb: (t * NB + nb, 0)),
            pl.BlockSpec((1, H), lambda t, nb: (0, 0)),
            pl.BlockSpec((H, H), lambda t, nb: (0, 0)),
        ],
        out_specs=pl.BlockSpec((BN, H), lambda t, nb: (t * NB + nb, 0)),
        out_shape=jax.ShapeDtypeStruct((T * N_PAD, H), jnp.float32),
    )(y1, p_flat, p_flat, dis, b1, W2)


def _gru_cell(xt, h, Wih, Whh, bih, bhh):
    gi = lax.dot_general(xt, Wih, (((1,), (1,)), ((), ())),
                         preferred_element_type=jnp.float32) + bih
    gh = lax.dot_general(h, Whh, (((1,), (1,)), ((), ())),
                         preferred_element_type=jnp.float32) + bhh
    r = jax.nn.sigmoid(gi[:, :H] + gh[:, :H])
    z = jax.nn.sigmoid(gi[:, H:2 * H] + gh[:, H:2 * H])
    n = jnp.tanh(gi[:, 2 * H:] + r * gh[:, 2 * H:])
    return (1.0 - z) * n + z * h


def _k3_body(y_ref, q0_ref, q1_ref, dis_ref, b2_ref, batch_ref,
             wih0_ref, whh0_ref, bih0_ref, bhh0_ref,
             wih1_ref, whh1_ref, bih1_ref, bhh1_ref,
             wc_ref, bc_ref, out_ref, acc_ref, cnt_ref):
    t = pl.program_id(0)
    nb = pl.program_id(1)

    @pl.when(jnp.logical_and(t == 0, nb == 0))
    def _init():
        acc_ref[...] = jnp.zeros_like(acc_ref)
        cnt_ref[...] = jnp.zeros_like(cnt_ref)

    dis = dis_ref[...]
    h2 = (q0_ref[...] + q1_ref[...] - y_ref[...]) * dis + b2_ref[...]
    h2 = jnp.maximum(h2, 0.0)
    bvals = batch_ref[0, 0, :]
    onehot = (bvals[:, None] == lax.broadcasted_iota(jnp.int32, (1, G), 1)
              ).astype(jnp.float32)
    pool = lax.dot_general(onehot, h2, (((0,), (0,)), ((), ())),
                           preferred_element_type=jnp.float32)
    acc_ref[pl.ds(t, 1)] = acc_ref[pl.ds(t, 1)] + pool[None]

    @pl.when(t == 0)
    def _cnt():
        ones = jnp.ones((BN, 1), jnp.float32)
        cnt_ref[...] += lax.dot_general(
            onehot, ones, (((0,), (0,)), ((), ())),
            preferred_element_type=jnp.float32)

    @pl.when(jnp.logical_and(t == T - 1, nb == NB - 1))
    def _final():
        cnt = jnp.maximum(cnt_ref[...], 1.0)
        seq = acc_ref[...] / cnt[None]
        h = jnp.zeros((G, H), jnp.float32)
        seq1 = []
        for tt in range(T):
            h = _gru_cell(seq[tt], h, wih0_ref[...], whh0_ref[...],
                          bih0_ref[...], bhh0_ref[...])
            seq1.append(h)
        h = jnp.zeros((G, H), jnp.float32)
        for tt in range(T):
            h = _gru_cell(seq1[tt], h, wih1_ref[...], whh1_ref[...],
                          bih1_ref[...], bhh1_ref[...])
        out_ref[...] = jnp.dot(h, wc_ref[...],
                               preferred_element_type=jnp.float32) + bc_ref[...]


def _tc_k3(y2, q_flat, dis, b2, batch_r, Wih0, Whh0, bih0, bhh0,
           Wih1, Whh1, bih1, bhh1, Wc, bc):
    full = lambda t, nb: (0, 0)
    return pl.pallas_call(
        _k3_body,
        grid=(T, NB),
        in_specs=[
            pl.BlockSpec((BN, H), lambda t, nb: (t * NB + nb, 0)),
            pl.BlockSpec((BN, H), lambda t, nb: ((t * 2 + 0) * NB + nb, 0)),
            pl.BlockSpec((BN, H), lambda t, nb: ((t * 2 + 1) * NB + nb, 0)),
            pl.BlockSpec((BN, 1), lambda t, nb: (t * NB + nb, 0)),
            pl.BlockSpec((1, H), full),
            pl.BlockSpec((1, 1, BN), lambda t, nb: (nb, 0, 0)),
            pl.BlockSpec((3 * H, H), full),
            pl.BlockSpec((3 * H, H), full),
            pl.BlockSpec((1, 3 * H), full),
            pl.BlockSpec((1, 3 * H), full),
            pl.BlockSpec((3 * H, H), full),
            pl.BlockSpec((3 * H, H), full),
            pl.BlockSpec((1, 3 * H), full),
            pl.BlockSpec((1, 3 * H), full),
            pl.BlockSpec((H, C), full),
            pl.BlockSpec((1, C), full),
        ],
        out_specs=pl.BlockSpec((G, C), full),
        out_shape=jax.ShapeDtypeStruct((G, C), jnp.float32),
        scratch_shapes=[
            pltpu.VMEM((T, G, H), jnp.float32),
            pltpu.VMEM((G, 1), jnp.float32),
        ],
    )(y2, q_flat, q_flat, dis, b2, batch_r, Wih0, Whh0, bih0, bhh0,
      Wih1, Whh1, bih1, bhh1, Wc, bc)


def kernel(x, edge_index, batch, W1, b1, W2, b2, Wih0, Whh0, bih0, bhh0,
           Wih1, Whh1, bih1, bhh1, Wc, bc):
    i32 = jnp.int32
    x_flat = jnp.pad(x, ((0, 0), (0, N_PAD - N), (0, 0))).reshape(T * N_PAD, D)

    src = edge_index[:, 0, :]
    dst = edge_index[:, 1, :]
    fill = (N + jnp.arange(E_PAD - E, dtype=i32) % (N_PAD - N))[None, :]
    fill = jnp.broadcast_to(fill, (T, E_PAD - E))
    srcp = jnp.concatenate([src, fill], axis=1)
    dstp = jnp.concatenate([dst, fill], axis=1)
    toff = (jnp.arange(T, dtype=i32) * N_PAD)[:, None]
    srcf = (srcp + toff).reshape(T * NW * NCH, CH)
    dstf = dstp.reshape(T * NW * NCH, CH)
    batch_r = jnp.pad(batch, (0, N_PAD - N),
                      constant_values=G).reshape(NB, 1, BN)

    deg_flat = _sc_deg()(dstf).reshape(T * 2 * N_PAD, 1)
    y1, dis = _tc_k1(x_flat, deg_flat, W1)
    p_flat = _sc_scatter()(y1, srcf, dstf)
    y2 = _tc_k2(y1, p_flat, dis, b1.reshape(1, H), W2)
    q_flat = _sc_scatter()(y2, srcf, dstf)
    return _tc_k3(y2, q_flat, dis, b2.reshape(1, H), batch_r,
                  Wih0, Whh0, bih0.reshape(1, -1), bhh0.reshape(1, -1),
                  Wih1, Whh1, bih1.reshape(1, -1), bhh1.reshape(1, -1),
                  Wc, bc.reshape(1, -1))
